# baseline (speedup 1.0000x reference)
import functools
import math

import jax
import jax.numpy as jnp
from jax import lax
from jax.experimental import pallas as pl
from jax.experimental.pallas import tpu as pltpu
from jax.experimental.pallas import tpu_sc as plsc

_NC = 2
_NS = 16
_LANE = 16
_NW = _NC * _NS

_SKIP_W = 0.005
_BN_EPS = 1e-5
_INV_SQRT = 1.0 / math.sqrt(1.0 + _BN_EPS)

_f32 = jnp.float32
_i32 = jnp.int32


@functools.lru_cache(maxsize=None)
def _make_deg(N, E, C):
    EW = E // _NW
    NCH = EW // C
    mesh = plsc.VectorSubcoreMesh(core_axis_name="c", subcore_axis_name="s")

    def deg_body(col_hbm, out_hbm, colbuf, degbuf):
        c = lax.axis_index("c")
        s = lax.axis_index("s")
        wid = s * _NC + c
        pltpu.sync_copy(col_hbm.at[wid], colbuf)
        z16 = jnp.zeros((_LANE,), _f32)
        o16 = jnp.ones((_LANE,), _f32)

        def zb(i, carry):
            degbuf[pl.ds(i * _LANE, _LANE)] = z16
            return carry

        lax.fori_loop(0, N // _LANE, zb, 0)
        kv = C // _LANE

        def sb(i, carry):
            r = i // kv
            k = i - r * kv
            idx = colbuf[r, pl.ds(k * _LANE, _LANE)]
            plsc.addupdate_scatter(degbuf, [idx], o16)
            return carry

        lax.fori_loop(0, EW // _LANE, sb, 0)
        pltpu.sync_copy(degbuf, out_hbm.at[wid, 0])

    return pl.kernel(
        deg_body,
        out_type=jax.ShapeDtypeStruct((_NW, 1, N), _f32),
        mesh=mesh,
        scratch_types=[
            pltpu.VMEM((NCH, C), _i32),
            pltpu.VMEM((N,), _f32),
        ],
        compiler_params=pltpu.CompilerParams(needs_layout_passes=False, use_tc_tiling_on_sc=False),
    )


@functools.lru_cache(maxsize=None)
def _make_hop(N, E, D, C):
    DH = D // 2
    EW = E // _NS
    NCH = EW // C
    RT = 624
    RZ = 208
    TAIL = N - RT * _NS
    NB = 4
    mesh = plsc.VectorSubcoreMesh(core_axis_name="c", subcore_axis_name="s")

    def hop_body(taba_hbm, tabb_hbm, row_hbm, col_hbm, dinv2_hbm,
                 outa_hbm, outb_hbm,
                 rowbuf, colbuf, g0, g1, g2, g3, zbuf, dvbuf, acc,
                 sg0, sg1, sg2, sg3, ss0, ss1, ss2, ss3):
        gbufs = (g0, g1, g2, g3)
        gsems = (sg0, sg1, sg2, sg3)
        ssems = (ss0, ss1, ss2, ss3)
        c = lax.axis_index("c")
        s = lax.axis_index("s")
        pltpu.sync_copy(row_hbm.at[s], rowbuf)
        pltpu.sync_copy(col_hbm.at[s], colbuf)
        pltpu.sync_copy(dinv2_hbm.at[pl.ds(s * RT, RT)],
                        dvbuf.at[pl.ds(0, RT)])

        @pl.when(s == _NS - 1)
        def _load_dv_tail():
            pltpu.sync_copy(dinv2_hbm.at[pl.ds(RT * _NS, TAIL)],
                            dvbuf.at[pl.ds(RT, TAIL)])

        z16 = jnp.zeros((_LANE,), _f32)
        nvec = DH // _LANE

        def zb(i, carry):
            r = i // nvec
            k = i - r * nvec
            zbuf[r, pl.ds(k * _LANE, _LANE)] = z16
            return carry

        lax.fori_loop(0, RZ * nvec, zb, 0)
        for t in range(RT // RZ):
            pltpu.sync_copy(zbuf, acc.at[pl.ds(s * RT + t * RZ, RZ)])

        @pl.when(s == _NS - 1)
        def _zero_tail():
            pltpu.sync_copy(zbuf.at[pl.ds(0, TAIL)],
                            acc.at[pl.ds(RT * _NS, TAIL)])

        plsc.subcore_barrier()

        def sweep(tab_hbm):
            for b in range(NB - 1):
                pltpu.async_copy(tab_hbm.at[rowbuf.at[b]], gbufs[b],
                                 gsems[b])

            def body(jj, carry):
                for b in range(NB):
                    j = jj * NB + b
                    pltpu.make_async_copy(
                        tab_hbm.at[rowbuf.at[j]], gbufs[b], gsems[b]).wait()
                    pltpu.async_copy(gbufs[b], acc.at[colbuf.at[j]],
                                     ssems[b], add=True)
                    nb2 = (b + NB - 1) % NB

                    @pl.when(j + NB - 1 < NCH)
                    def _issue_next():
                        @pl.when(j >= 1)
                        def _wait_prev_scatter():
                            pltpu.make_async_copy(
                                gbufs[nb2], acc.at[colbuf.at[j - 1]],
                                ssems[nb2]).wait()

                        pltpu.async_copy(tab_hbm.at[rowbuf.at[j + NB - 1]],
                                         gbufs[nb2], gsems[nb2])

                return carry

            lax.fori_loop(0, NCH // NB, body, 0)
            for b in range(NB):
                pltpu.make_async_copy(
                    gbufs[b], acc.at[colbuf.at[NCH - NB + b]],
                    ssems[b]).wait()

        @pl.when(c == 0)
        def _sweep_a():
            sweep(taba_hbm)

        @pl.when(c == 1)
        def _sweep_b():
            sweep(tabb_hbm)

        plsc.subcore_barrier()

        def scale_rows(nrows, dv_off):
            def rb(r, carry):
                dvs = dvbuf[pl.ds(dv_off + r, _LANE)][0]
                for k in range(nvec):
                    sl = pl.ds(k * _LANE, _LANE)
                    zbuf[r, sl] = zbuf[r, sl] * dvs
                return carry

            lax.fori_loop(0, nrows, rb, 0)

        for t in range(RT // RZ):
            off = s * RT + t * RZ
            pltpu.sync_copy(acc.at[pl.ds(off, RZ)], zbuf)
            scale_rows(RZ, t * RZ)

            @pl.when(c == 0)
            def _wa():
                pltpu.sync_copy(zbuf, outa_hbm.at[pl.ds(off, RZ)])

            @pl.when(c == 1)
            def _wb():
                pltpu.sync_copy(zbuf, outb_hbm.at[pl.ds(off, RZ)])

        @pl.when(s == _NS - 1)
        def _scale_tail():
            off = RT * _NS
            pltpu.sync_copy(acc.at[pl.ds(off, TAIL)], zbuf.at[pl.ds(0, TAIL)])
            scale_rows(TAIL, RT)

            @pl.when(c == 0)
            def _wa():
                pltpu.sync_copy(zbuf.at[pl.ds(0, TAIL)],
                                outa_hbm.at[pl.ds(off, TAIL)])

            @pl.when(c == 1)
            def _wb():
                pltpu.sync_copy(zbuf.at[pl.ds(0, TAIL)],
                                outb_hbm.at[pl.ds(off, TAIL)])

    return pl.kernel(
        hop_body,
        out_type=(jax.ShapeDtypeStruct((N, DH), _f32),
                  jax.ShapeDtypeStruct((N, DH), _f32)),
        mesh=mesh,
        scratch_types=[
            pltpu.VMEM((NCH, C), _i32),
            pltpu.VMEM((NCH, C), _i32),
            pltpu.VMEM((C, DH), _f32),
            pltpu.VMEM((C, DH), _f32),
            pltpu.VMEM((C, DH), _f32),
            pltpu.VMEM((C, DH), _f32),
            pltpu.VMEM((RZ, DH), _f32),
            pltpu.VMEM((RT + 32,), _f32),
            pltpu.VMEM_SHARED((N, DH), _f32),
        ] + [pltpu.SemaphoreType.DMA] * 8,
        compiler_params=pltpu.CompilerParams(needs_layout_passes=False, use_tc_tiling_on_sc=False),
    )


@functools.lru_cache(maxsize=None)
def _make_pool_gather(N, D, M):
    mesh = plsc.VectorSubcoreMesh(core_axis_name="c", subcore_axis_name="s")
    CH = M // _NC

    def pg_body(h_hbm, idx_hbm, out_hbm, idxbuf, rows, sem):
        c = lax.axis_index("c")
        s = lax.axis_index("s")

        @pl.when(s == 0)
        def _run():
            base = c * CH
            pltpu.sync_copy(idx_hbm.at[pl.ds(base, CH)], idxbuf)
            pltpu.async_copy(h_hbm.at[idxbuf], rows, sem).wait()
            pltpu.sync_copy(rows, out_hbm.at[pl.ds(base, CH)])

    return pl.kernel(
        pg_body,
        out_type=jax.ShapeDtypeStruct((M, D), _f32),
        mesh=mesh,
        scratch_types=[
            pltpu.VMEM((CH,), _i32),
            pltpu.VMEM((CH, D), _f32),
            pltpu.SemaphoreType.DMA,
        ],
        compiler_params=pltpu.CompilerParams(needs_layout_passes=False, use_tc_tiling_on_sc=False),
    )


def _dinv_body(p_ref, o_ref, o2_ref, os_ref):
    deg = jnp.sum(p_ref[...], axis=0, keepdims=True)
    pos = deg > 0
    dinv = jnp.where(pos, lax.rsqrt(deg), 0.0)
    o_ref[...] = dinv
    o2_ref[...] = dinv * dinv
    os_ref[...] = jnp.where(pos, jnp.sqrt(deg), 0.0)


def _tc_dinv(partials):
    NWp, N = partials.shape
    sds = jax.ShapeDtypeStruct((1, N), _f32)
    return pl.pallas_call(
        _dinv_body,
        out_shape=[sds, sds, sds],
    )(partials)


def _scale_body(x_ref, d_ref, oa_ref, ob_ref):
    DH = oa_ref.shape[1]
    hp = x_ref[...] * d_ref[...]
    oa_ref[...] = hp[:, :DH]
    ob_ref[...] = hp[:, DH:]


def _tc_scale(x, dinv):
    N, D = x.shape
    DH = D // 2
    R = 1000
    halfspec = pl.BlockSpec((R, DH), lambda i: (i, 0))
    hp_a, hp_b = pl.pallas_call(
        _scale_body,
        grid=(N // R,),
        in_specs=[pl.BlockSpec((R, D), lambda i: (i, 0)),
                  pl.BlockSpec((R, 1), lambda i: (i, 0))],
        out_specs=[halfspec, halfspec],
        out_shape=[jax.ShapeDtypeStruct((N, DH), _f32),
                   jax.ShapeDtypeStruct((N, DH), _f32)],
    )(x, dinv)
    return hp_a, hp_b


def _h0_body(h_ref, w0_ref, bias_ref, o_ref):
    o_ref[...] = (jnp.dot(h_ref[...], w0_ref[...], preferred_element_type=_f32)
                  + bias_ref[...])


def _tc_h0(h, W0, bias):
    N, D = h.shape
    H = bias.shape[0]
    R = 1000
    return pl.pallas_call(
        _h0_body,
        grid=(N // R,),
        in_specs=[pl.BlockSpec((R, D), lambda i: (i, 0)),
                  pl.BlockSpec((D, H), lambda i: (0, 0)),
                  pl.BlockSpec((1, H), lambda i: (0, 0))],
        out_specs=pl.BlockSpec((R, H), lambda i: (i, 0)),
        out_shape=jax.ShapeDtypeStruct((N, H), _f32),
    )(h, W0.T, bias.reshape(1, H))


def _end_body(h0_ref, ga_ref, gb_ref, sq_ref, b0_ref, b1_ref, d_ref,
              w1_ref, w2_ref,
              gwd_ref, gbd_ref, g0_ref, gd_ref, bs_ref,
              hn_ref, hpa_ref, hpb_ref):
    DH = b0_ref.shape[1]
    dv = d_ref[...]
    a1 = jnp.concatenate([ga_ref[...], gb_ref[...]], axis=1) * sq_ref[...]
    a2 = jnp.concatenate([b0_ref[...], b1_ref[...]], axis=1) * dv
    out = (h0_ref[...]
           + jnp.dot(a1, w1_ref[...], preferred_element_type=_f32)
           + jnp.dot(a2, w2_ref[...], preferred_element_type=_f32))
    hr = jnp.maximum(out, 0.0)
    dl = jnp.sum(hr * gwd_ref[...], axis=1, keepdims=True) + gbd_ref[...]
    edl = jnp.exp(-jnp.abs(dl))
    s1 = jnp.where(dl >= 0, 1.0 / (1.0 + edl), edl / (1.0 + edl))
    coef = (g0_ref[...] + s1 * gd_ref[...]) * _INV_SQRT
    hn = hr + _SKIP_W * (hr * coef + bs_ref[...])
    hn_ref[...] = hn
    hp = hn * dv
    hpa_ref[...] = hp[:, :DH]
    hpb_ref[...] = hp[:, DH:]


def _tc_end(h0, ga, gb_, sqrtdeg, acca, accbb, dinv, W, gw, gb,
            gamma, beta):
    N, D = h0.shape
    DH = D // 2
    H = D
    w1t, w2t = W[1].T, W[2].T
    gwd = gw[1:2] - gw[0:1]
    gbd = (gb[1] - gb[0]).reshape(1, 1)
    g0row = gamma[:H].reshape(1, H)
    gdrow = (gamma[H:] - gamma[:H]).reshape(1, H)
    bsrow = (beta[:H] + beta[H:]).reshape(1, H)
    R = 1000
    rowspec = pl.BlockSpec((R, D), lambda i: (i, 0))
    halfspec = pl.BlockSpec((R, DH), lambda i: (i, 0))
    wspec = pl.BlockSpec((D, H), lambda i: (0, 0))
    brow = pl.BlockSpec((1, H), lambda i: (0, 0))
    hn, hp_a, hp_b = pl.pallas_call(
        _end_body,
        grid=(N // R,),
        in_specs=[rowspec,
                  halfspec, halfspec,
                  pl.BlockSpec((R, 1), lambda i: (i, 0)),
                  halfspec, halfspec,
                  pl.BlockSpec((R, 1), lambda i: (i, 0)),
                  wspec, wspec,
                  brow, pl.BlockSpec((1, 1), lambda i: (0, 0)),
                  brow, brow, brow],
        out_specs=[rowspec, halfspec, halfspec],
        out_shape=[jax.ShapeDtypeStruct((N, H), _f32),
                   jax.ShapeDtypeStruct((N, DH), _f32),
                   jax.ShapeDtypeStruct((N, DH), _f32)],
    )(h0, ga, gb_, sqrtdeg, acca, accbb, dinv, w1t, w2t,
      gwd, gbd, g0row, gdrow, bsrow)
    return hn, hp_a, hp_b


def _tc_head(emb, G, params):
    H = params['merger_b'].shape[0]
    OUTD = params['ffn_b2'].shape[0]
    mT = params['merger_W'].T
    m0t, m1t, m2t = mT[:H], mT[H:2 * H], mT[2 * H:]
    f1t = params['ffn_W1'].T
    f2t = params['ffn_W2'].T
    mb = params['merger_b'].reshape(1, H)
    fb1 = params['ffn_b1'].reshape(1, H)
    fb2 = params['ffn_b2'].reshape(1, OUTD)
    PH = emb.shape[0] // 2

    def head_body(emb_ref, m0_ref, m1_ref, m2_ref, mb_ref,
                  f1_ref, fb1_ref, f2_ref, fb2_ref, o_ref):
        em = emb_ref[...]
        e0 = em[0:PH]
        e1 = em[PH:2 * PH]
        d = jnp.abs(e0 - e1)
        mn = (e0 + e1) * 0.5
        mx = jnp.maximum(e0, e1)
        pooled = (jnp.dot(d, m0_ref[...], preferred_element_type=_f32)
                  + jnp.dot(mn, m1_ref[...], preferred_element_type=_f32)
                  + jnp.dot(mx, m2_ref[...], preferred_element_type=_f32)
                  + mb_ref[...])
        hid = jnp.maximum(
            jnp.dot(pooled, f1_ref[...], preferred_element_type=_f32)
            + fb1_ref[...], 0.0)
        res = (jnp.dot(hid, f2_ref[...], preferred_element_type=_f32)
               + fb2_ref[...])
        o_ref[...] = res[0:G]

    return pl.pallas_call(
        head_body,
        out_shape=jax.ShapeDtypeStruct((G, OUTD), _f32),
    )(emb, m0t, m1t, m2t, mb, f1t, fb1, f2t, fb2)


def kernel(x, edge_index, batch, set_indices, num_graphs, params):
    del batch, num_graphs
    N, D = x.shape
    E = edge_index.shape[1]
    G = set_indices.shape[0]
    row = edge_index[0]
    col = edge_index[1]

    C = 125
    row_s = row.reshape(_NS, (E // _NS) // C, C)
    col_s = col.reshape(_NS, (E // _NS) // C, C)
    CD = 80
    col_w = col.reshape(_NW, (E // _NW) // CD, CD)

    deg_parts = _make_deg(N, E, CD)(col_w).reshape(_NW, N)
    dinv_row, dinv2_row, sq_row = _tc_dinv(deg_parts)
    dinv = dinv_row.reshape(N, 1)
    dinv2 = dinv2_row.reshape(N)
    sqrtdeg = sq_row.reshape(N, 1)

    hop = _make_hop(N, E, D, C)
    ones_n = jnp.ones((N,), _f32)
    hp_a, hp_b = _tc_scale(x, dinv)
    h = x
    for l in range(3):
        h0 = _tc_h0(h, params['tag%d_W' % l][0], params['tag%d_b' % l])
        g_a, g_b = hop(hp_a, hp_b, row_s, col_s, dinv2)
        acc_a, acc_b = hop(g_a, g_b, row_s, col_s, ones_n)
        h, hp_a, hp_b = _tc_end(h0, g_a, g_b, sqrtdeg, acc_a, acc_b, dinv,
                                params['tag%d_W' % l],
                                params['gn%d_gw' % l], params['gn%d_gb' % l],
                                params['gn%d_gamma' % l],
                                params['gn%d_beta' % l])

    seg = N // G
    bases = jnp.arange(G, dtype=_i32) * seg
    idx2 = bases[:, None] + set_indices.astype(_i32)
    PH = 128
    pad = jnp.zeros((PH - G,), _i32)
    idx_flat = jnp.concatenate([idx2[:, 0], pad, idx2[:, 1], pad])
    emb = _make_pool_gather(N, D, 2 * PH)(h, idx_flat)
    return _tc_head(emb, G, params)


# speedup vs baseline: 1.0057x; 1.0057x over previous
import functools
import math

import jax
import jax.numpy as jnp
from jax import lax
from jax.experimental import pallas as pl
from jax.experimental.pallas import tpu as pltpu
from jax.experimental.pallas import tpu_sc as plsc

_NC = 2
_NS = 16
_LANE = 16
_NW = _NC * _NS

_SKIP_W = 0.005
_BN_EPS = 1e-5
_INV_SQRT = 1.0 / math.sqrt(1.0 + _BN_EPS)

_f32 = jnp.float32
_i32 = jnp.int32


@functools.lru_cache(maxsize=None)
def _make_deg(N, E, C):
    EW = E // _NW
    NCH = EW // C
    mesh = plsc.VectorSubcoreMesh(core_axis_name="c", subcore_axis_name="s")

    def deg_body(col_hbm, out_hbm, colbuf, degbuf):
        c = lax.axis_index("c")
        s = lax.axis_index("s")
        wid = s * _NC + c
        pltpu.sync_copy(col_hbm.at[wid], colbuf)
        z16 = jnp.zeros((_LANE,), _f32)
        o16 = jnp.ones((_LANE,), _f32)

        def zb(i, carry):
            degbuf[pl.ds(i * _LANE, _LANE)] = z16
            return carry

        lax.fori_loop(0, N // _LANE, zb, 0)
        kv = C // _LANE

        def sb(i, carry):
            r = i // kv
            k = i - r * kv
            idx = colbuf[r, pl.ds(k * _LANE, _LANE)]
            plsc.addupdate_scatter(degbuf, [idx], o16)
            return carry

        lax.fori_loop(0, EW // _LANE, sb, 0)
        pltpu.sync_copy(degbuf, out_hbm.at[wid, 0])

    return pl.kernel(
        deg_body,
        out_type=jax.ShapeDtypeStruct((_NW, 1, N), _f32),
        mesh=mesh,
        scratch_types=[
            pltpu.VMEM((NCH, C), _i32),
            pltpu.VMEM((N,), _f32),
        ],
        compiler_params=pltpu.CompilerParams(needs_layout_passes=False, use_tc_tiling_on_sc=False),
    )


@functools.lru_cache(maxsize=None)
def _make_hop(N, E, D, C):
    DH = D // 2
    EW = E // _NS
    NCH = EW // C
    RT = 624
    RZ = 208
    TAIL = N - RT * _NS
    NB = 4
    mesh = plsc.VectorSubcoreMesh(core_axis_name="c", subcore_axis_name="s")

    def hop_body(taba_hbm, tabb_hbm, row_hbm, col_hbm, dinv2_hbm,
                 outa_hbm, outb_hbm,
                 rowbuf, colbuf, g0, g1, g2, g3, zbuf, dvbuf, acc,
                 sg0, sg1, sg2, sg3, ss0, ss1, ss2, ss3):
        gbufs = (g0, g1, g2, g3)
        gsems = (sg0, sg1, sg2, sg3)
        ssems = (ss0, ss1, ss2, ss3)
        c = lax.axis_index("c")
        s = lax.axis_index("s")
        pltpu.sync_copy(row_hbm.at[s], rowbuf)
        pltpu.sync_copy(col_hbm.at[s], colbuf)
        pltpu.sync_copy(dinv2_hbm.at[pl.ds(s * RT, RT)],
                        dvbuf.at[pl.ds(0, RT)])

        @pl.when(s == _NS - 1)
        def _load_dv_tail():
            pltpu.sync_copy(dinv2_hbm.at[pl.ds(RT * _NS, TAIL)],
                            dvbuf.at[pl.ds(RT, TAIL)])

        z16 = jnp.zeros((_LANE,), _f32)
        nvec = DH // _LANE

        def zb(i, carry):
            r = i // nvec
            k = i - r * nvec
            zbuf[r, pl.ds(k * _LANE, _LANE)] = z16
            return carry

        lax.fori_loop(0, RZ * nvec, zb, 0)
        for t in range(RT // RZ):
            pltpu.sync_copy(zbuf, acc.at[pl.ds(s * RT + t * RZ, RZ)])

        @pl.when(s == _NS - 1)
        def _zero_tail():
            pltpu.sync_copy(zbuf.at[pl.ds(0, TAIL)],
                            acc.at[pl.ds(RT * _NS, TAIL)])

        plsc.subcore_barrier()

        def sweep(tab_hbm):
            for b in range(NB - 1):
                pltpu.async_copy(tab_hbm.at[rowbuf.at[b]], gbufs[b],
                                 gsems[b])

            def body(jj, carry):
                for b in range(NB):
                    j = jj * NB + b
                    pltpu.make_async_copy(
                        tab_hbm.at[rowbuf.at[j]], gbufs[b], gsems[b]).wait()
                    pltpu.async_copy(gbufs[b], acc.at[colbuf.at[j]],
                                     ssems[b], add=True)
                    nb2 = (b + NB - 1) % NB

                    @pl.when(j + NB - 1 < NCH)
                    def _issue_next():
                        @pl.when(j >= 1)
                        def _wait_prev_scatter():
                            pltpu.make_async_copy(
                                gbufs[nb2], acc.at[colbuf.at[j - 1]],
                                ssems[nb2]).wait()

                        pltpu.async_copy(tab_hbm.at[rowbuf.at[j + NB - 1]],
                                         gbufs[nb2], gsems[nb2])

                return carry

            lax.fori_loop(0, NCH // NB, body, 0)
            for b in range(NB):
                pltpu.make_async_copy(
                    gbufs[b], acc.at[colbuf.at[NCH - NB + b]],
                    ssems[b]).wait()

        @pl.when(c == 0)
        def _sweep_a():
            sweep(taba_hbm)

        @pl.when(c == 1)
        def _sweep_b():
            sweep(tabb_hbm)

        plsc.subcore_barrier()

        def scale_rows(nrows, dv_off):
            def rb(r, carry):
                dvs = dvbuf[pl.ds(dv_off + r, _LANE)][0]
                for k in range(nvec):
                    sl = pl.ds(k * _LANE, _LANE)
                    zbuf[r, sl] = zbuf[r, sl] * dvs
                return carry

            lax.fori_loop(0, nrows, rb, 0)

        for t in range(RT // RZ):
            off = s * RT + t * RZ
            pltpu.sync_copy(acc.at[pl.ds(off, RZ)], zbuf)
            scale_rows(RZ, t * RZ)

            @pl.when(c == 0)
            def _wa():
                pltpu.sync_copy(zbuf, outa_hbm.at[pl.ds(off, RZ)])

            @pl.when(c == 1)
            def _wb():
                pltpu.sync_copy(zbuf, outb_hbm.at[pl.ds(off, RZ)])

        @pl.when(s == _NS - 1)
        def _scale_tail():
            off = RT * _NS
            pltpu.sync_copy(acc.at[pl.ds(off, TAIL)], zbuf.at[pl.ds(0, TAIL)])
            scale_rows(TAIL, RT)

            @pl.when(c == 0)
            def _wa():
                pltpu.sync_copy(zbuf.at[pl.ds(0, TAIL)],
                                outa_hbm.at[pl.ds(off, TAIL)])

            @pl.when(c == 1)
            def _wb():
                pltpu.sync_copy(zbuf.at[pl.ds(0, TAIL)],
                                outb_hbm.at[pl.ds(off, TAIL)])

    return pl.kernel(
        hop_body,
        out_type=(jax.ShapeDtypeStruct((N, DH), _f32),
                  jax.ShapeDtypeStruct((N, DH), _f32)),
        mesh=mesh,
        scratch_types=[
            pltpu.VMEM((NCH, C), _i32),
            pltpu.VMEM((NCH, C), _i32),
            pltpu.VMEM((C, DH), _f32),
            pltpu.VMEM((C, DH), _f32),
            pltpu.VMEM((C, DH), _f32),
            pltpu.VMEM((C, DH), _f32),
            pltpu.VMEM((RZ, DH), _f32),
            pltpu.VMEM((RT + 32,), _f32),
            pltpu.VMEM_SHARED((N, DH), _f32),
        ] + [pltpu.SemaphoreType.DMA] * 8,
        compiler_params=pltpu.CompilerParams(needs_layout_passes=False, use_tc_tiling_on_sc=False),
    )


@functools.lru_cache(maxsize=None)
def _make_pool_gather(N, D, M):
    mesh = plsc.VectorSubcoreMesh(core_axis_name="c", subcore_axis_name="s")
    CH = M // _NC

    def pg_body(h_hbm, idx_hbm, out_hbm, idxbuf, rows, sem):
        c = lax.axis_index("c")
        s = lax.axis_index("s")

        @pl.when(s == 0)
        def _run():
            base = c * CH
            pltpu.sync_copy(idx_hbm.at[pl.ds(base, CH)], idxbuf)
            pltpu.async_copy(h_hbm.at[idxbuf], rows, sem).wait()
            pltpu.sync_copy(rows, out_hbm.at[pl.ds(base, CH)])

    return pl.kernel(
        pg_body,
        out_type=jax.ShapeDtypeStruct((M, D), _f32),
        mesh=mesh,
        scratch_types=[
            pltpu.VMEM((CH,), _i32),
            pltpu.VMEM((CH, D), _f32),
            pltpu.SemaphoreType.DMA,
        ],
        compiler_params=pltpu.CompilerParams(needs_layout_passes=False, use_tc_tiling_on_sc=False),
    )


def _dinv_body(p_ref, o_ref, o2_ref, os_ref):
    deg = jnp.sum(p_ref[...], axis=0, keepdims=True)
    pos = deg > 0
    dinv = jnp.where(pos, lax.rsqrt(deg), 0.0)
    o_ref[...] = dinv
    o2_ref[...] = dinv * dinv
    os_ref[...] = jnp.where(pos, jnp.sqrt(deg), 0.0)


def _tc_dinv(partials):
    NWp, N = partials.shape
    sds = jax.ShapeDtypeStruct((1, N), _f32)
    return pl.pallas_call(
        _dinv_body,
        out_shape=[sds, sds, sds],
    )(partials)


def _scale_body(x_ref, d_ref, oa_ref, ob_ref):
    DH = oa_ref.shape[1]
    hp = x_ref[...] * d_ref[...]
    oa_ref[...] = hp[:, :DH]
    ob_ref[...] = hp[:, DH:]


def _tc_scale(x, dinv):
    N, D = x.shape
    DH = D // 2
    R = 1000
    halfspec = pl.BlockSpec((R, DH), lambda i: (i, 0))
    hp_a, hp_b = pl.pallas_call(
        _scale_body,
        grid=(N // R,),
        in_specs=[pl.BlockSpec((R, D), lambda i: (i, 0)),
                  pl.BlockSpec((R, 1), lambda i: (i, 0))],
        out_specs=[halfspec, halfspec],
        out_shape=[jax.ShapeDtypeStruct((N, DH), _f32),
                   jax.ShapeDtypeStruct((N, DH), _f32)],
    )(x, dinv)
    return hp_a, hp_b


def _end_body(h_ref, ga_ref, gb_ref, sq_ref, b0_ref, b1_ref, d_ref,
              w0_ref, w1_ref, w2_ref,
              bias_ref, gwd_ref, gbd_ref, g0_ref, gd_ref, bs_ref,
              hn_ref, hpa_ref, hpb_ref):
    DH = b0_ref.shape[1]
    dv = d_ref[...]
    a1 = jnp.concatenate([ga_ref[...], gb_ref[...]], axis=1) * sq_ref[...]
    a2 = jnp.concatenate([b0_ref[...], b1_ref[...]], axis=1) * dv
    out = (jnp.dot(h_ref[...], w0_ref[...], preferred_element_type=_f32)
           + jnp.dot(a1, w1_ref[...], preferred_element_type=_f32)
           + jnp.dot(a2, w2_ref[...], preferred_element_type=_f32)
           + bias_ref[...])
    hr = jnp.maximum(out, 0.0)
    dl = jnp.sum(hr * gwd_ref[...], axis=1, keepdims=True) + gbd_ref[...]
    edl = jnp.exp(-jnp.abs(dl))
    s1 = jnp.where(dl >= 0, 1.0 / (1.0 + edl), edl / (1.0 + edl))
    coef = (g0_ref[...] + s1 * gd_ref[...]) * _INV_SQRT
    hn = hr + _SKIP_W * (hr * coef + bs_ref[...])
    hn_ref[...] = hn
    hp = hn * dv
    hpa_ref[...] = hp[:, :DH]
    hpb_ref[...] = hp[:, DH:]


def _tc_end(h, ga, gb_, sqrtdeg, acca, accbb, dinv, W, bias, gw, gb,
            gamma, beta):
    N, D = h.shape
    DH = D // 2
    H = bias.shape[0]
    w0t, w1t, w2t = W[0].T, W[1].T, W[2].T
    bias_row = bias.reshape(1, H)
    gwd = gw[1:2] - gw[0:1]
    gbd = (gb[1] - gb[0]).reshape(1, 1)
    g0row = gamma[:H].reshape(1, H)
    gdrow = (gamma[H:] - gamma[:H]).reshape(1, H)
    bsrow = (beta[:H] + beta[H:]).reshape(1, H)
    R = 1000
    rowspec = pl.BlockSpec((R, D), lambda i: (i, 0))
    halfspec = pl.BlockSpec((R, DH), lambda i: (i, 0))
    wspec = pl.BlockSpec((D, H), lambda i: (0, 0))
    brow = pl.BlockSpec((1, H), lambda i: (0, 0))
    hn, hp_a, hp_b = pl.pallas_call(
        _end_body,
        grid=(N // R,),
        in_specs=[rowspec,
                  halfspec, halfspec,
                  pl.BlockSpec((R, 1), lambda i: (i, 0)),
                  halfspec, halfspec,
                  pl.BlockSpec((R, 1), lambda i: (i, 0)),
                  wspec, wspec, wspec,
                  brow, brow, pl.BlockSpec((1, 1), lambda i: (0, 0)),
                  brow, brow, brow],
        out_specs=[rowspec, halfspec, halfspec],
        out_shape=[jax.ShapeDtypeStruct((N, H), _f32),
                   jax.ShapeDtypeStruct((N, DH), _f32),
                   jax.ShapeDtypeStruct((N, DH), _f32)],
    )(h, ga, gb_, sqrtdeg, acca, accbb, dinv, w0t, w1t, w2t,
      bias_row, gwd, gbd, g0row, gdrow, bsrow)
    return hn, hp_a, hp_b


def _tc_head(emb, G, params):
    H = params['merger_b'].shape[0]
    OUTD = params['ffn_b2'].shape[0]
    mT = params['merger_W'].T
    m0t, m1t, m2t = mT[:H], mT[H:2 * H], mT[2 * H:]
    f1t = params['ffn_W1'].T
    f2t = params['ffn_W2'].T
    mb = params['merger_b'].reshape(1, H)
    fb1 = params['ffn_b1'].reshape(1, H)
    fb2 = params['ffn_b2'].reshape(1, OUTD)
    PH = emb.shape[0] // 2

    def head_body(emb_ref, m0_ref, m1_ref, m2_ref, mb_ref,
                  f1_ref, fb1_ref, f2_ref, fb2_ref, o_ref):
        em = emb_ref[...]
        e0 = em[0:PH]
        e1 = em[PH:2 * PH]
        d = jnp.abs(e0 - e1)
        mn = (e0 + e1) * 0.5
        mx = jnp.maximum(e0, e1)
        pooled = (jnp.dot(d, m0_ref[...], preferred_element_type=_f32)
                  + jnp.dot(mn, m1_ref[...], preferred_element_type=_f32)
                  + jnp.dot(mx, m2_ref[...], preferred_element_type=_f32)
                  + mb_ref[...])
        hid = jnp.maximum(
            jnp.dot(pooled, f1_ref[...], preferred_element_type=_f32)
            + fb1_ref[...], 0.0)
        res = (jnp.dot(hid, f2_ref[...], preferred_element_type=_f32)
               + fb2_ref[...])
        o_ref[...] = res[0:G]

    return pl.pallas_call(
        head_body,
        out_shape=jax.ShapeDtypeStruct((G, OUTD), _f32),
    )(emb, m0t, m1t, m2t, mb, f1t, fb1, f2t, fb2)


def kernel(x, edge_index, batch, set_indices, num_graphs, params):
    del batch, num_graphs
    N, D = x.shape
    E = edge_index.shape[1]
    G = set_indices.shape[0]
    row = edge_index[0]
    col = edge_index[1]

    C = 125
    row_s = row.reshape(_NS, (E // _NS) // C, C)
    col_s = col.reshape(_NS, (E // _NS) // C, C)
    CD = 80
    col_w = col.reshape(_NW, (E // _NW) // CD, CD)

    deg_parts = _make_deg(N, E, CD)(col_w).reshape(_NW, N)
    dinv_row, dinv2_row, sq_row = _tc_dinv(deg_parts)
    dinv = dinv_row.reshape(N, 1)
    dinv2 = dinv2_row.reshape(N)
    sqrtdeg = sq_row.reshape(N, 1)

    hop = _make_hop(N, E, D, C)
    ones_n = jnp.ones((N,), _f32)
    hp_a, hp_b = _tc_scale(x, dinv)
    h = x
    for l in range(3):
        g_a, g_b = hop(hp_a, hp_b, row_s, col_s, dinv2)
        acc_a, acc_b = hop(g_a, g_b, row_s, col_s, ones_n)
        h, hp_a, hp_b = _tc_end(h, g_a, g_b, sqrtdeg, acc_a, acc_b, dinv,
                                params['tag%d_W' % l], params['tag%d_b' % l],
                                params['gn%d_gw' % l], params['gn%d_gb' % l],
                                params['gn%d_gamma' % l],
                                params['gn%d_beta' % l])

    seg = N // G
    bases = jnp.arange(G, dtype=_i32) * seg
    idx2 = bases[:, None] + set_indices.astype(_i32)
    PH = 128
    pad = jnp.zeros((PH - G,), _i32)
    idx_flat = jnp.concatenate([idx2[:, 0], pad, idx2[:, 1], pad])
    emb = _make_pool_gather(N, D, 2 * PH)(h, idx_flat)
    return _tc_head(emb, G, params)
